# baseline (speedup 1.0000x reference)
import functools

import jax
import jax.numpy as jnp
from jax import lax
from jax.experimental import pallas as pl
from jax.experimental.pallas import tpu as pltpu
from jax.experimental.pallas import tpu_sc as plsc

N = 10000
E = 320000
D = 128
NC = 2
NS = 16
NW = NC * NS
EP = E // NW
K = 128
EPP = 10240
CH = EPP // K
PAD = EPP - EP
NP = 10240
RC = 128
RPS = NP // NS
BR = 1000
R = N // BR

_MESH = plsc.VectorSubcoreMesh(core_axis_name="c", subcore_axis_name="s")


def _sc_degree(dst0, dst1):
    @functools.partial(
        pl.kernel,
        mesh=_MESH,
        out_type=jax.ShapeDtypeStruct((NC, 2, NP, D), jnp.float32),
        scratch_types=[
            pltpu.VMEM((CH, K), jnp.int32),
            pltpu.VMEM((K, D), jnp.float32),
            pltpu.VMEM((RC, D), jnp.float32),
            pltpu.VMEM_SHARED((NP, D), jnp.float32),
            pltpu.SemaphoreType.DMA,
        ],
    )
    def k(dst0_hbm, dst1_hbm, out_hbm, idx_v, ones_v, stg_v, dg, sem):
        cid = lax.axis_index("c")
        sid = lax.axis_index("s")
        wid = sid * NC + cid

        def fill1(i, _):
            for c in range(D // 16):
                ones_v[i, pl.ds(c * 16, 16)] = jnp.ones((16,), jnp.float32)
            return _

        lax.fori_loop(0, RC, fill1, None)

        for s, dh in enumerate((dst0_hbm, dst1_hbm)):
            def fill(i, _):
                for c in range(D // 16):
                    stg_v[i, pl.ds(c * 16, 16)] = jnp.zeros((16,),
                                                            jnp.float32)
                return _

            lax.fori_loop(0, RC, fill, None)
            for t in range(RPS // RC):
                r0 = sid * RPS + t * RC
                pltpu.sync_copy(stg_v, dg.at[pl.ds(r0, RC)])
            plsc.subcore_barrier()
            pltpu.sync_copy(dh.at[wid], idx_v)

            def body(i, _):
                descs = [pltpu.async_copy(ones_v, dg.at[idx_v.at[i * 8 + b]],
                                          sem, add=True) for b in range(8)]
                for d_ in descs:
                    d_.wait()
                return _

            lax.fori_loop(0, CH // 8, body, None)
            plsc.subcore_barrier()

            for t in range(RPS // RC):
                r0 = sid * RPS + t * RC
                pltpu.sync_copy(dg.at[pl.ds(r0, RC)], stg_v)
                pltpu.sync_copy(stg_v, out_hbm.at[cid, s, pl.ds(r0, RC)])
            plsc.subcore_barrier()

    return k(dst0, dst1)


def _sc_scatter(hp, packed):
    NB = 2
    RND = CH // NB

    @functools.partial(
        pl.kernel,
        mesh=_MESH,
        out_type=jax.ShapeDtypeStruct((NC, NP, D), jnp.float32),
        scratch_types=[
            pltpu.VMEM((CH, K), jnp.int32),
            pltpu.VMEM((NB, K), jnp.int32),
            pltpu.VMEM((NB, K), jnp.int32),
            pltpu.VMEM((NB, K, D), jnp.float32),
            pltpu.VMEM_SHARED((NP, D), jnp.float32),
        ] + [pltpu.SemaphoreType.DMA] * (2 * NB),
    )
    def k(h_hbm, pk_hbm, out_hbm, pk_v, src_t, dst_t, rows_v, acc, *sems):
        gsem = sems[:NB]
        ssem = sems[NB:]
        cid = lax.axis_index("c")
        sid = lax.axis_index("s")
        wid = sid * NC + cid

        def fill(i, _):
            for c in range(D // 16):
                rows_v[0, i, pl.ds(c * 16, 16)] = jnp.zeros((16,),
                                                            jnp.float32)
            return _

        lax.fori_loop(0, RC, fill, None)
        for t in range(RPS // RC):
            r0 = sid * RPS + t * RC
            pltpu.sync_copy(rows_v.at[0], acc.at[pl.ds(r0, RC)])
        plsc.subcore_barrier()

        pltpu.sync_copy(pk_hbm.at[wid], pk_v)

        def unp_src(j, b):
            for c in range(K // 16):
                v = pk_v[j, pl.ds(c * 16, 16)]
                src_t[b, pl.ds(c * 16, 16)] = lax.shift_right_logical(v, 14)

        def unp_dst(j, b):
            for c in range(K // 16):
                v = pk_v[j, pl.ds(c * 16, 16)]
                dst_t[b, pl.ds(c * 16, 16)] = lax.bitwise_and(v, 16383)

        def g_issue(b):
            pltpu.async_copy(h_hbm.at[src_t.at[b]], rows_v.at[b], gsem[b])

        def g_wait(b):
            pltpu.make_async_copy(h_hbm.at[src_t.at[b]], rows_v.at[b],
                                  gsem[b]).wait()

        def s_issue(b):
            pltpu.async_copy(rows_v.at[b], acc.at[dst_t.at[b]], ssem[b],
                             add=True)

        def s_wait(b):
            pltpu.make_async_copy(rows_v.at[b], acc.at[dst_t.at[b]],
                                  ssem[b]).wait()

        for b in range(NB):
            unp_src(b, b)
            g_issue(b)

        def round_body(i, _):
            j0 = i * NB
            for b in range(NB):
                g_wait(b)
                unp_dst(j0 + b, b)
                s_issue(b)
            for b in range(NB):
                unp_src(j0 + NB + b, b)
                s_wait(b)
                g_issue(b)
            return _

        lax.fori_loop(0, RND - 1, round_body, None)
        j0 = (RND - 1) * NB
        for b in range(NB):
            g_wait(b)
            unp_dst(j0 + b, b)
            s_issue(b)
        for b in range(NB):
            s_wait(b)
        plsc.subcore_barrier()

        for t in range(RPS // RC):
            r0 = sid * RPS + t * RC
            pltpu.sync_copy(acc.at[pl.ds(r0, RC)], rows_v.at[0])
            pltpu.sync_copy(rows_v.at[0], out_hbm.at[cid, pl.ds(r0, RC)])

    return k(hp, packed)


def _dot(a, b):
    return jnp.dot(a, b, preferred_element_type=jnp.float32)


def _accsum(ar):
    a = ar[...]
    return a[0] + a[1]


def _ln(x, g, b):
    m = jnp.mean(x, axis=-1, keepdims=True)
    v = jnp.mean((x - m) * (x - m), axis=-1, keepdims=True)
    return (x - m) * lax.rsqrt(v + 1e-5) * g + b


def _transformer(x0, x1, wqkv, bqkv, wout, bout, wf1, bf1, wf2, bf2,
                 l1g, l1b, l2g, l2b):
    qkv0 = _dot(x0, wqkv) + bqkv
    qkv1 = _dot(x1, wqkv) + bqkv
    q0, k0, v0 = qkv0[:, 0:D], qkv0[:, D:2 * D], qkv0[:, 2 * D:3 * D]
    q1, k1, v1 = qkv1[:, 0:D], qkv1[:, D:2 * D], qkv1[:, 2 * D:3 * D]
    scale = 1.0 / (D ** 0.5)
    s00 = jnp.sum(q0 * k0, axis=-1, keepdims=True) * scale
    s01 = jnp.sum(q0 * k1, axis=-1, keepdims=True) * scale
    s10 = jnp.sum(q1 * k0, axis=-1, keepdims=True) * scale
    s11 = jnp.sum(q1 * k1, axis=-1, keepdims=True) * scale
    m0 = jnp.maximum(s00, s01)
    e00 = jnp.exp(s00 - m0)
    e01 = jnp.exp(s01 - m0)
    m1 = jnp.maximum(s10, s11)
    e10 = jnp.exp(s10 - m1)
    e11 = jnp.exp(s11 - m1)
    a0 = (e00 * v0 + e01 * v1) / (e00 + e01)
    a1 = (e10 * v0 + e11 * v1) / (e10 + e11)
    o0 = _dot(a0, wout) + bout
    o1 = _dot(a1, wout) + bout
    y0 = _ln(x0 + o0, l1g, l1b)
    y1 = _ln(x1 + o1, l1g, l1b)
    f0 = _dot(jnp.maximum(_dot(y0, wf1) + bf1, 0.0), wf2) + bf2
    f1 = _dot(jnp.maximum(_dot(y1, wf1) + bf1, 0.0), wf2) + bf2
    z0 = _ln(y0 + f0, l2g, l2b)
    z1 = _ln(y1 + f1, l2g, l2b)
    return 0.5 * (z0 + z1)


def _rb(shape=(BR, D)):
    return pl.BlockSpec(shape, lambda r: (r,) + (0,) * (len(shape) - 1))


def _full(shape):
    return pl.BlockSpec(shape, lambda r: (0,) * len(shape))


def _tc_prep(deg_parts, x, w):
    def body(dp, xr, wr, d0r, d1r, hpr):
        dp_ = dp[...]
        d0 = lax.rsqrt(dp_[0, 0][:, :1] + dp_[1, 0][:, :1] + 1.0)
        d1 = lax.rsqrt(dp_[0, 1][:, :1] + dp_[1, 1][:, :1] + 1.0)
        d0r[...] = d0
        d1r[...] = d1
        hpr[...] = _dot(xr[...], wr[...]) * d0

    return pl.pallas_call(
        body,
        grid=(R,),
        in_specs=[
            pl.BlockSpec((NC, 2, BR, D), lambda r: (0, 0, r, 0)),
            _rb(), _full((D, D)),
        ],
        out_specs=[_rb((BR, 1)), _rb((BR, 1)), _rb()],
        out_shape=[
            jax.ShapeDtypeStruct((N, 1), jnp.float32),
            jax.ShapeDtypeStruct((N, 1), jnp.float32),
            jax.ShapeDtypeStruct((N, D), jnp.float32),
        ],
    )(deg_parts, x, w)


def _tc_gcn_next(accp, hp, dinv, b, wn):
    def body(ar, hr, dr, br, wr, tr, nr):
        d = dr[...]
        tilde = jnp.maximum((_accsum(ar) + hr[...]) * d + br[...], 0.0)
        tr[...] = tilde
        nr[...] = _dot(tilde, wr[...]) * d

    return pl.pallas_call(
        body,
        grid=(R,),
        in_specs=[
            pl.BlockSpec((NC, BR, D), lambda r: (0, r, 0)),
            _rb(), _rb((BR, 1)), _full((1, D)), _full((D, D)),
        ],
        out_specs=[_rb(), _rb()],
        out_shape=[
            jax.ShapeDtypeStruct((N, D), jnp.float32),
            jax.ShapeDtypeStruct((N, D), jnp.float32),
        ],
    )(accp, hp, dinv, b, wn)


def _tc_gcn_bridge(accp, hp, dinv0, b, x1, w0, dinv1):
    def body(ar, hr, d0r, br, xr, wr, d1r, tr, gr):
        tilde = jnp.maximum((_accsum(ar) + hr[...]) * d0r[...] + br[...], 0.0)
        tr[...] = tilde
        gr[...] = _dot(xr[...], wr[...]) * d1r[...]

    return pl.pallas_call(
        body,
        grid=(R,),
        in_specs=[
            pl.BlockSpec((NC, BR, D), lambda r: (0, r, 0)),
            _rb(), _rb((BR, 1)), _full((1, D)), _rb(), _full((D, D)),
            _rb((BR, 1)),
        ],
        out_specs=[_rb(), _rb()],
        out_shape=[
            jax.ShapeDtypeStruct((N, D), jnp.float32),
            jax.ShapeDtypeStruct((N, D), jnp.float32),
        ],
    )(accp, hp, dinv0, b, x1, w0, dinv1)


def _tc_trans(accp, gp, dinv, bg, prev, tp, wn):
    def body(ar, gr, dr, bgr, pr, wqkv, bqkv, wout, bout, wf1, bf1, wf2, bf2,
             l1g, l1b, l2g, l2b, wnr, outr):
        d = dr[...]
        tilde = jnp.maximum((_accsum(ar) + gr[...]) * d + bgr[...], 0.0)
        h = _transformer(pr[...], tilde, wqkv[...], bqkv[...], wout[...],
                         bout[...], wf1[...], bf1[...], wf2[...], bf2[...],
                         l1g[...], l1b[...], l2g[...], l2b[...])
        outr[...] = _dot(h, wnr[...]) * d

    return pl.pallas_call(
        body,
        grid=(R,),
        in_specs=[
            pl.BlockSpec((NC, BR, D), lambda r: (0, r, 0)),
            _rb(), _rb((BR, 1)), _full((1, D)), _rb(),
            _full((D, 3 * D)), _full((1, 3 * D)), _full((D, D)),
            _full((1, D)), _full((D, 2048)), _full((1, 2048)),
            _full((2048, D)), _full((1, D)),
            _full((1, D)), _full((1, D)), _full((1, D)), _full((1, D)),
            _full((D, D)),
        ],
        out_specs=[_rb()],
        out_shape=[jax.ShapeDtypeStruct((N, D), jnp.float32)],
    )(accp, gp, dinv, bg, prev, *tp, wn)[0]


def _tc_final(accp, gp, dinv, bg, prev, tp, mlp_wt, mlp_b, out_wt, out_b):
    def body(ar, gr, dr, bgr, pr, wqkv, bqkv, wout, bout, wf1, bf1, wf2, bf2,
             l1g, l1b, l2g, l2b, mwr, mbr, owr, obr, outr, pacc):
        r = pl.program_id(0)
        d = dr[...]
        tilde = jnp.maximum((_accsum(ar) + gr[...]) * d + bgr[...], 0.0)
        h = _transformer(pr[...], tilde, wqkv[...], bqkv[...], wout[...],
                         bout[...], wf1[...], bf1[...], wf2[...], bf2[...],
                         l1g[...], l1b[...], l2g[...], l2b[...])

        @pl.when(r == 0)
        def _():
            pacc[...] = jnp.zeros_like(pacc)

        pacc[...] += jnp.sum(h, axis=0, keepdims=True)
        pooled = pacc[...] * (1.0 / N)
        hm = jnp.maximum(_dot(pooled, mwr[...]) + mbr[...], 0.0)
        outr[...] = _dot(hm, owr[...]) + obr[...]

    return pl.pallas_call(
        body,
        grid=(R,),
        in_specs=[
            pl.BlockSpec((NC, BR, D), lambda r: (0, r, 0)),
            _rb(), _rb((BR, 1)), _full((1, D)), _rb(),
            _full((D, 3 * D)), _full((1, 3 * D)), _full((D, D)),
            _full((1, D)), _full((D, 2048)), _full((1, 2048)),
            _full((2048, D)), _full((1, D)),
            _full((1, D)), _full((1, D)), _full((1, D)), _full((1, D)),
            _full((D, D)), _full((1, D)), _full((D, 16)), _full((1, 16)),
        ],
        out_specs=[_full((1, 16))],
        out_shape=[jax.ShapeDtypeStruct((1, 16), jnp.float32)],
        scratch_shapes=[pltpu.VMEM((1, D), jnp.float32)],
    )(accp, gp, dinv, bg, prev, *tp, mlp_wt, mlp_b, out_wt, out_b)[0]


def _tparams(lp):
    return (
        lp['in_proj_w'].T, lp['in_proj_b'][None, :],
        lp['out_proj_w'].T, lp['out_proj_b'][None, :],
        lp['lin1_w'].T, lp['lin1_b'][None, :],
        lp['lin2_w'].T, lp['lin2_b'][None, :],
        lp['ln1_g'][None, :], lp['ln1_b'][None, :],
        lp['ln2_g'][None, :], lp['ln2_b'][None, :],
    )


def kernel(x0, x1, edge_index0, edge_index1, params):
    def prep_idx(row, fill):
        r = row.astype(jnp.int32).reshape(NW, EP)
        return jnp.pad(r, ((0, 0), (0, PAD)),
                       constant_values=fill).reshape(NW, CH, K)

    src0 = prep_idx(edge_index0[0], 0)
    dst0 = prep_idx(edge_index0[1], N)
    src1 = prep_idx(edge_index1[0], 0)
    dst1 = prep_idx(edge_index1[1], N)

    l0, l1 = params['layers']
    w0, b0 = l0['gcn_W'], l0['gcn_b'][None, :]
    w1, b1 = l1['gcn_W'], l1['gcn_b'][None, :]

    deg_parts = _sc_degree(dst0, dst1)
    dinv0, dinv1, h0p = _tc_prep(deg_parts, x0, w0)

    pk0 = jnp.sort(((src0 << 14) | dst0).reshape(NW, EPP),
                   axis=1).reshape(NW, CH, K)
    pk1 = jnp.sort(((src1 << 14) | dst1).reshape(NW, EPP),
                   axis=1).reshape(NW, CH, K)

    acc = _sc_scatter(h0p, pk0)
    prev0, h1p = _tc_gcn_next(acc, h0p, dinv0, b0, w1)
    acc = _sc_scatter(h1p, pk0)
    prev1, g0p = _tc_gcn_bridge(acc, h1p, dinv0, b1, x1, w0, dinv1)

    acc = _sc_scatter(g0p, pk1)
    g1p = _tc_trans(acc, g0p, dinv1, b0, prev0, _tparams(l0), w1)
    acc = _sc_scatter(g1p, pk1)
    return _tc_final(acc, g1p, dinv1, b1, prev1, _tparams(l1),
                     params['mlp_w'].T, params['mlp_b'][None, :],
                     params['out_w'].T, params['out_b'][None, :])


# speedup vs baseline: 1.0812x; 1.0812x over previous
import functools

import jax
import jax.numpy as jnp
from jax import lax
from jax.experimental import pallas as pl
from jax.experimental.pallas import tpu as pltpu
from jax.experimental.pallas import tpu_sc as plsc

N = 10000
E = 320000
D = 128
NC = 2
NS = 16
NW = NC * NS
EP = E // NW
K = 128
EPP = 10240
CH = EPP // K
PAD = EPP - EP
NP = 10240
RC = 128
RPS = NP // NS
BR = 1000
R = N // BR

_MESH = plsc.VectorSubcoreMesh(core_axis_name="c", subcore_axis_name="s")


def _sc_degree(dst0, dst1):
    @functools.partial(
        pl.kernel,
        mesh=_MESH,
        out_type=jax.ShapeDtypeStruct((NC, 2, NP, D), jnp.float32),
        scratch_types=[
            pltpu.VMEM((CH, K), jnp.int32),
            pltpu.VMEM((K, D), jnp.float32),
            pltpu.VMEM((RC, D), jnp.float32),
            pltpu.VMEM_SHARED((NP, D), jnp.float32),
            pltpu.SemaphoreType.DMA,
        ],
    )
    def k(dst0_hbm, dst1_hbm, out_hbm, idx_v, ones_v, stg_v, dg, sem):
        cid = lax.axis_index("c")
        sid = lax.axis_index("s")
        wid = sid * NC + cid

        def fill1(i, _):
            for c in range(D // 16):
                ones_v[i, pl.ds(c * 16, 16)] = jnp.ones((16,), jnp.float32)
            return _

        lax.fori_loop(0, RC, fill1, None)

        for s, dh in enumerate((dst0_hbm, dst1_hbm)):
            def fill(i, _):
                for c in range(D // 16):
                    stg_v[i, pl.ds(c * 16, 16)] = jnp.zeros((16,),
                                                            jnp.float32)
                return _

            lax.fori_loop(0, RC, fill, None)
            for t in range(RPS // RC):
                r0 = sid * RPS + t * RC
                pltpu.sync_copy(stg_v, dg.at[pl.ds(r0, RC)])
            plsc.subcore_barrier()
            pltpu.sync_copy(dh.at[wid], idx_v)

            def body(i, _):
                descs = [pltpu.async_copy(ones_v, dg.at[idx_v.at[i * 8 + b]],
                                          sem, add=True) for b in range(8)]
                for d_ in descs:
                    d_.wait()
                return _

            lax.fori_loop(0, CH // 8, body, None)
            plsc.subcore_barrier()

            for t in range(RPS // RC):
                r0 = sid * RPS + t * RC
                pltpu.sync_copy(dg.at[pl.ds(r0, RC)], stg_v)
                pltpu.sync_copy(stg_v, out_hbm.at[cid, s, pl.ds(r0, RC)])
            plsc.subcore_barrier()

    return k(dst0, dst1)


def _sc_scatter(hp, packed):
    NB = 2
    RND = CH // NB

    @functools.partial(
        pl.kernel,
        mesh=_MESH,
        out_type=jax.ShapeDtypeStruct((NC, NP, D), jnp.float32),
        scratch_types=[
            pltpu.VMEM((CH, K), jnp.int32),
            pltpu.VMEM((NB, K), jnp.int32),
            pltpu.VMEM((NB, K), jnp.int32),
            pltpu.VMEM((NB, K, D), jnp.float32),
            pltpu.VMEM_SHARED((NP, D), jnp.float32),
        ] + [pltpu.SemaphoreType.DMA] * (2 * NB),
    )
    def k(h_hbm, pk_hbm, out_hbm, pk_v, src_t, dst_t, rows_v, acc, *sems):
        gsem = sems[:NB]
        ssem = sems[NB:]
        cid = lax.axis_index("c")
        sid = lax.axis_index("s")
        wid = sid * NC + cid

        def fill(i, _):
            for c in range(D // 16):
                rows_v[0, i, pl.ds(c * 16, 16)] = jnp.zeros((16,),
                                                            jnp.float32)
            return _

        lax.fori_loop(0, RC, fill, None)
        for t in range(RPS // RC):
            r0 = sid * RPS + t * RC
            pltpu.sync_copy(rows_v.at[0], acc.at[pl.ds(r0, RC)])
        plsc.subcore_barrier()

        pltpu.sync_copy(pk_hbm.at[wid], pk_v)

        def unp_src(j, b):
            for c in range(K // 16):
                v = pk_v[j, pl.ds(c * 16, 16)]
                src_t[b, pl.ds(c * 16, 16)] = lax.shift_right_logical(v, 14)

        def unp_dst(j, b):
            for c in range(K // 16):
                v = pk_v[j, pl.ds(c * 16, 16)]
                dst_t[b, pl.ds(c * 16, 16)] = lax.bitwise_and(v, 16383)

        def g_issue(b):
            pltpu.async_copy(h_hbm.at[src_t.at[b]], rows_v.at[b], gsem[b])

        def g_wait(b):
            pltpu.make_async_copy(h_hbm.at[src_t.at[b]], rows_v.at[b],
                                  gsem[b]).wait()

        def s_issue(b):
            pltpu.async_copy(rows_v.at[b], acc.at[dst_t.at[b]], ssem[b],
                             add=True)

        def s_wait(b):
            pltpu.make_async_copy(rows_v.at[b], acc.at[dst_t.at[b]],
                                  ssem[b]).wait()

        for b in range(NB):
            unp_src(b, b)
            g_issue(b)

        def round_body(i, _):
            j0 = i * NB
            for b in range(NB):
                g_wait(b)
                unp_dst(j0 + b, b)
                s_issue(b)
            for b in range(NB):
                unp_src(j0 + NB + b, b)
                s_wait(b)
                g_issue(b)
            return _

        lax.fori_loop(0, RND - 1, round_body, None)
        j0 = (RND - 1) * NB
        for b in range(NB):
            g_wait(b)
            unp_dst(j0 + b, b)
            s_issue(b)
        for b in range(NB):
            s_wait(b)
        plsc.subcore_barrier()

        for t in range(RPS // RC):
            r0 = sid * RPS + t * RC
            pltpu.sync_copy(acc.at[pl.ds(r0, RC)], rows_v.at[0])
            pltpu.sync_copy(rows_v.at[0], out_hbm.at[cid, pl.ds(r0, RC)])

    return k(hp, packed)


def _dot(a, b):
    return jnp.dot(a, b, preferred_element_type=jnp.float32)


def _bdot(a, b):
    return jnp.dot(a.astype(jnp.bfloat16), b,
                   preferred_element_type=jnp.float32)


def _accsum(ar):
    a = ar[...]
    return a[0] + a[1]


def _ln(x, g, b):
    m = jnp.mean(x, axis=-1, keepdims=True)
    v = jnp.mean((x - m) * (x - m), axis=-1, keepdims=True)
    return (x - m) * lax.rsqrt(v + 1e-5) * g + b


def _transformer(x0, x1, wqkv, bqkv, wout, bout, wf1, bf1, wf2, bf2,
                 l1g, l1b, l2g, l2b):
    qkv0 = _bdot(x0, wqkv) + bqkv
    qkv1 = _bdot(x1, wqkv) + bqkv
    q0, k0, v0 = qkv0[:, 0:D], qkv0[:, D:2 * D], qkv0[:, 2 * D:3 * D]
    q1, k1, v1 = qkv1[:, 0:D], qkv1[:, D:2 * D], qkv1[:, 2 * D:3 * D]
    scale = 1.0 / (D ** 0.5)
    s00 = jnp.sum(q0 * k0, axis=-1, keepdims=True) * scale
    s01 = jnp.sum(q0 * k1, axis=-1, keepdims=True) * scale
    s10 = jnp.sum(q1 * k0, axis=-1, keepdims=True) * scale
    s11 = jnp.sum(q1 * k1, axis=-1, keepdims=True) * scale
    m0 = jnp.maximum(s00, s01)
    e00 = jnp.exp(s00 - m0)
    e01 = jnp.exp(s01 - m0)
    m1 = jnp.maximum(s10, s11)
    e10 = jnp.exp(s10 - m1)
    e11 = jnp.exp(s11 - m1)
    a0 = (e00 * v0 + e01 * v1) / (e00 + e01)
    a1 = (e10 * v0 + e11 * v1) / (e10 + e11)
    o0 = _bdot(a0, wout) + bout
    o1 = _bdot(a1, wout) + bout
    y0 = _ln(x0 + o0, l1g, l1b)
    y1 = _ln(x1 + o1, l1g, l1b)
    f0 = _bdot(jnp.maximum(_bdot(y0, wf1) + bf1, 0.0), wf2) + bf2
    f1 = _bdot(jnp.maximum(_bdot(y1, wf1) + bf1, 0.0), wf2) + bf2
    z0 = _ln(y0 + f0, l2g, l2b)
    z1 = _ln(y1 + f1, l2g, l2b)
    return 0.5 * (z0 + z1)


def _rb(shape=(BR, D)):
    return pl.BlockSpec(shape, lambda r: (r,) + (0,) * (len(shape) - 1))


def _full(shape):
    return pl.BlockSpec(shape, lambda r: (0,) * len(shape))


def _tc_prep(deg_parts, x, w):
    def body(dp, xr, wr, d0r, d1r, hpr):
        dp_ = dp[...]
        d0 = lax.rsqrt(dp_[0, 0][:, :1] + dp_[1, 0][:, :1] + 1.0)
        d1 = lax.rsqrt(dp_[0, 1][:, :1] + dp_[1, 1][:, :1] + 1.0)
        d0r[...] = d0
        d1r[...] = d1
        hpr[...] = _dot(xr[...], wr[...]) * d0

    return pl.pallas_call(
        body,
        grid=(R,),
        in_specs=[
            pl.BlockSpec((NC, 2, BR, D), lambda r: (0, 0, r, 0)),
            _rb(), _full((D, D)),
        ],
        out_specs=[_rb((BR, 1)), _rb((BR, 1)), _rb()],
        out_shape=[
            jax.ShapeDtypeStruct((N, 1), jnp.float32),
            jax.ShapeDtypeStruct((N, 1), jnp.float32),
            jax.ShapeDtypeStruct((N, D), jnp.float32),
        ],
    )(deg_parts, x, w)


def _tc_gcn_next(accp, hp, dinv, b, wn):
    def body(ar, hr, dr, br, wr, tr, nr):
        d = dr[...]
        tilde = jnp.maximum((_accsum(ar) + hr[...]) * d + br[...], 0.0)
        tr[...] = tilde
        nr[...] = _dot(tilde, wr[...]) * d

    return pl.pallas_call(
        body,
        grid=(R,),
        in_specs=[
            pl.BlockSpec((NC, BR, D), lambda r: (0, r, 0)),
            _rb(), _rb((BR, 1)), _full((1, D)), _full((D, D)),
        ],
        out_specs=[_rb(), _rb()],
        out_shape=[
            jax.ShapeDtypeStruct((N, D), jnp.float32),
            jax.ShapeDtypeStruct((N, D), jnp.float32),
        ],
    )(accp, hp, dinv, b, wn)


def _tc_gcn_bridge(accp, hp, dinv0, b, x1, w0, dinv1):
    def body(ar, hr, d0r, br, xr, wr, d1r, tr, gr):
        tilde = jnp.maximum((_accsum(ar) + hr[...]) * d0r[...] + br[...], 0.0)
        tr[...] = tilde
        gr[...] = _dot(xr[...], wr[...]) * d1r[...]

    return pl.pallas_call(
        body,
        grid=(R,),
        in_specs=[
            pl.BlockSpec((NC, BR, D), lambda r: (0, r, 0)),
            _rb(), _rb((BR, 1)), _full((1, D)), _rb(), _full((D, D)),
            _rb((BR, 1)),
        ],
        out_specs=[_rb(), _rb()],
        out_shape=[
            jax.ShapeDtypeStruct((N, D), jnp.float32),
            jax.ShapeDtypeStruct((N, D), jnp.float32),
        ],
    )(accp, hp, dinv0, b, x1, w0, dinv1)


def _tc_trans(accp, gp, dinv, bg, prev, tp, wn):
    def body(ar, gr, dr, bgr, pr, wqkv, bqkv, wout, bout, wf1, bf1, wf2, bf2,
             l1g, l1b, l2g, l2b, wnr, outr):
        d = dr[...]
        tilde = jnp.maximum((_accsum(ar) + gr[...]) * d + bgr[...], 0.0)
        h = _transformer(pr[...], tilde, wqkv[...], bqkv[...], wout[...],
                         bout[...], wf1[...], bf1[...], wf2[...], bf2[...],
                         l1g[...], l1b[...], l2g[...], l2b[...])
        outr[...] = _dot(h, wnr[...]) * d

    return pl.pallas_call(
        body,
        grid=(R,),
        in_specs=[
            pl.BlockSpec((NC, BR, D), lambda r: (0, r, 0)),
            _rb(), _rb((BR, 1)), _full((1, D)), _rb(),
            _full((D, 3 * D)), _full((1, 3 * D)), _full((D, D)),
            _full((1, D)), _full((D, 2048)), _full((1, 2048)),
            _full((2048, D)), _full((1, D)),
            _full((1, D)), _full((1, D)), _full((1, D)), _full((1, D)),
            _full((D, D)),
        ],
        out_specs=[_rb()],
        out_shape=[jax.ShapeDtypeStruct((N, D), jnp.float32)],
    )(accp, gp, dinv, bg, prev, *tp, wn)[0]


def _tc_final(accp, gp, dinv, bg, prev, tp, mlp_wt, mlp_b, out_wt, out_b):
    def body(ar, gr, dr, bgr, pr, wqkv, bqkv, wout, bout, wf1, bf1, wf2, bf2,
             l1g, l1b, l2g, l2b, mwr, mbr, owr, obr, outr, pacc):
        r = pl.program_id(0)
        d = dr[...]
        tilde = jnp.maximum((_accsum(ar) + gr[...]) * d + bgr[...], 0.0)
        h = _transformer(pr[...], tilde, wqkv[...], bqkv[...], wout[...],
                         bout[...], wf1[...], bf1[...], wf2[...], bf2[...],
                         l1g[...], l1b[...], l2g[...], l2b[...])

        @pl.when(r == 0)
        def _():
            pacc[...] = jnp.zeros_like(pacc)

        pacc[...] += jnp.sum(h, axis=0, keepdims=True)
        pooled = pacc[...] * (1.0 / N)
        hm = jnp.maximum(_dot(pooled, mwr[...]) + mbr[...], 0.0)
        outr[...] = _dot(hm, owr[...]) + obr[...]

    return pl.pallas_call(
        body,
        grid=(R,),
        in_specs=[
            pl.BlockSpec((NC, BR, D), lambda r: (0, r, 0)),
            _rb(), _rb((BR, 1)), _full((1, D)), _rb(),
            _full((D, 3 * D)), _full((1, 3 * D)), _full((D, D)),
            _full((1, D)), _full((D, 2048)), _full((1, 2048)),
            _full((2048, D)), _full((1, D)),
            _full((1, D)), _full((1, D)), _full((1, D)), _full((1, D)),
            _full((D, D)), _full((1, D)), _full((D, 16)), _full((1, 16)),
        ],
        out_specs=[_full((1, 16))],
        out_shape=[jax.ShapeDtypeStruct((1, 16), jnp.float32)],
        scratch_shapes=[pltpu.VMEM((1, D), jnp.float32)],
    )(accp, gp, dinv, bg, prev, *tp, mlp_wt, mlp_b, out_wt, out_b)[0]


def _tparams(lp):
    bf = jnp.bfloat16
    return (
        lp['in_proj_w'].T.astype(bf), lp['in_proj_b'][None, :],
        lp['out_proj_w'].T.astype(bf), lp['out_proj_b'][None, :],
        lp['lin1_w'].T.astype(bf), lp['lin1_b'][None, :],
        lp['lin2_w'].T.astype(bf), lp['lin2_b'][None, :],
        lp['ln1_g'][None, :], lp['ln1_b'][None, :],
        lp['ln2_g'][None, :], lp['ln2_b'][None, :],
    )


def kernel(x0, x1, edge_index0, edge_index1, params):
    def prep_idx(row, fill):
        r = row.astype(jnp.int32).reshape(NW, EP)
        return jnp.pad(r, ((0, 0), (0, PAD)),
                       constant_values=fill).reshape(NW, CH, K)

    src0 = prep_idx(edge_index0[0], 0)
    dst0 = prep_idx(edge_index0[1], N)
    src1 = prep_idx(edge_index1[0], 0)
    dst1 = prep_idx(edge_index1[1], N)

    l0, l1 = params['layers']
    w0, b0 = l0['gcn_W'], l0['gcn_b'][None, :]
    w1, b1 = l1['gcn_W'], l1['gcn_b'][None, :]

    deg_parts = _sc_degree(dst0, dst1)
    dinv0, dinv1, h0p = _tc_prep(deg_parts, x0, w0)

    pk0 = (src0 << 14) | dst0
    pk1 = (src1 << 14) | dst1

    acc = _sc_scatter(h0p, pk0)
    prev0, h1p = _tc_gcn_next(acc, h0p, dinv0, b0, w1)
    acc = _sc_scatter(h1p, pk0)
    prev1, g0p = _tc_gcn_bridge(acc, h1p, dinv0, b1, x1, w0, dinv1)

    acc = _sc_scatter(g0p, pk1)
    g1p = _tc_trans(acc, g0p, dinv1, b0, prev0, _tparams(l0), w1)
    acc = _sc_scatter(g1p, pk1)
    return _tc_final(acc, g1p, dinv1, b1, prev1, _tparams(l1),
                     params['mlp_w'].T, params['mlp_b'][None, :],
                     params['out_w'].T, params['out_b'][None, :])


# speedup vs baseline: 1.0839x; 1.0025x over previous
import functools

import jax
import jax.numpy as jnp
from jax import lax
from jax.experimental import pallas as pl
from jax.experimental.pallas import tpu as pltpu
from jax.experimental.pallas import tpu_sc as plsc

N = 10000
E = 320000
D = 128
NC = 2
NS = 16
NW = NC * NS
EP = E // NW
K = 128
EPP = 10240
CH = EPP // K
PAD = EPP - EP
NP = 10240
RC = 128
RPS = NP // NS
BR = 1000
R = N // BR

_MESH = plsc.VectorSubcoreMesh(core_axis_name="c", subcore_axis_name="s")


def _sc_degree(dst0, dst1):
    @functools.partial(
        pl.kernel,
        mesh=_MESH,
        out_type=jax.ShapeDtypeStruct((NC, 2, NP, D), jnp.float32),
        scratch_types=[
            pltpu.VMEM((CH, K), jnp.int32),
            pltpu.VMEM((K, D), jnp.float32),
            pltpu.VMEM((RC, D), jnp.float32),
            pltpu.VMEM_SHARED((NP, D), jnp.float32),
            pltpu.SemaphoreType.DMA,
        ],
    )
    def k(dst0_hbm, dst1_hbm, out_hbm, idx_v, ones_v, stg_v, dg, sem):
        cid = lax.axis_index("c")
        sid = lax.axis_index("s")
        wid = sid * NC + cid

        def fill1(i, _):
            for c in range(D // 16):
                ones_v[i, pl.ds(c * 16, 16)] = jnp.ones((16,), jnp.float32)
            return _

        lax.fori_loop(0, RC, fill1, None)

        for s, dh in enumerate((dst0_hbm, dst1_hbm)):
            def fill(i, _):
                for c in range(D // 16):
                    stg_v[i, pl.ds(c * 16, 16)] = jnp.zeros((16,),
                                                            jnp.float32)
                return _

            lax.fori_loop(0, RC, fill, None)
            for t in range(RPS // RC):
                r0 = sid * RPS + t * RC
                pltpu.sync_copy(stg_v, dg.at[pl.ds(r0, RC)])
            plsc.subcore_barrier()
            pltpu.sync_copy(dh.at[wid], idx_v)

            def body(i, _):
                descs = [pltpu.async_copy(ones_v, dg.at[idx_v.at[i * 16 + b]],
                                          sem, add=True) for b in range(16)]
                for d_ in descs:
                    d_.wait()
                return _

            lax.fori_loop(0, CH // 16, body, None)
            plsc.subcore_barrier()

            for t in range(RPS // RC):
                r0 = sid * RPS + t * RC
                pltpu.sync_copy(dg.at[pl.ds(r0, RC)], stg_v)
                pltpu.sync_copy(stg_v, out_hbm.at[cid, s, pl.ds(r0, RC)])
            plsc.subcore_barrier()

    return k(dst0, dst1)


def _sc_scatter(hp, packed):
    NB = 2
    RND = CH // NB

    @functools.partial(
        pl.kernel,
        mesh=_MESH,
        out_type=jax.ShapeDtypeStruct((NC, NP, D), jnp.float32),
        scratch_types=[
            pltpu.VMEM((CH, K), jnp.int32),
            pltpu.VMEM((NB, K), jnp.int32),
            pltpu.VMEM((NB, K), jnp.int32),
            pltpu.VMEM((NB, K, D), jnp.float32),
            pltpu.VMEM_SHARED((NP, D), jnp.float32),
        ] + [pltpu.SemaphoreType.DMA] * (2 * NB),
    )
    def k(h_hbm, pk_hbm, out_hbm, pk_v, src_t, dst_t, rows_v, acc, *sems):
        gsem = sems[:NB]
        ssem = sems[NB:]
        cid = lax.axis_index("c")
        sid = lax.axis_index("s")
        wid = sid * NC + cid

        def fill(i, _):
            for c in range(D // 16):
                rows_v[0, i, pl.ds(c * 16, 16)] = jnp.zeros((16,),
                                                            jnp.float32)
            return _

        lax.fori_loop(0, RC, fill, None)
        for t in range(RPS // RC):
            r0 = sid * RPS + t * RC
            pltpu.sync_copy(rows_v.at[0], acc.at[pl.ds(r0, RC)])
        plsc.subcore_barrier()

        pltpu.sync_copy(pk_hbm.at[wid], pk_v)

        def unp_src(j, b):
            for c in range(K // 16):
                v = pk_v[j, pl.ds(c * 16, 16)]
                src_t[b, pl.ds(c * 16, 16)] = lax.shift_right_logical(v, 14)

        def unp_dst(j, b):
            for c in range(K // 16):
                v = pk_v[j, pl.ds(c * 16, 16)]
                dst_t[b, pl.ds(c * 16, 16)] = lax.bitwise_and(v, 16383)

        def g_issue(b):
            pltpu.async_copy(h_hbm.at[src_t.at[b]], rows_v.at[b], gsem[b])

        def g_wait(b):
            pltpu.make_async_copy(h_hbm.at[src_t.at[b]], rows_v.at[b],
                                  gsem[b]).wait()

        def s_issue(b):
            pltpu.async_copy(rows_v.at[b], acc.at[dst_t.at[b]], ssem[b],
                             add=True)

        def s_wait(b):
            pltpu.make_async_copy(rows_v.at[b], acc.at[dst_t.at[b]],
                                  ssem[b]).wait()

        for b in range(NB):
            unp_src(b, b)
            g_issue(b)

        def round_body(i, _):
            j0 = i * NB
            for b in range(NB):
                g_wait(b)
                unp_dst(j0 + b, b)
                s_issue(b)
            for b in range(NB):
                unp_src(j0 + NB + b, b)
                s_wait(b)
                g_issue(b)
            return _

        lax.fori_loop(0, RND - 1, round_body, None)
        j0 = (RND - 1) * NB
        for b in range(NB):
            g_wait(b)
            unp_dst(j0 + b, b)
            s_issue(b)
        for b in range(NB):
            s_wait(b)
        plsc.subcore_barrier()

        pltpu.sync_copy(acc.at[pl.ds(sid * RPS, RPS)],
                        out_hbm.at[cid, pl.ds(sid * RPS, RPS)])

    return k(hp, packed)


def _dot(a, b):
    return jnp.dot(a, b, preferred_element_type=jnp.float32)


def _accsum(ar):
    a = ar[...]
    return a[0] + a[1]


def _ln(x, g, b):
    m = jnp.mean(x, axis=-1, keepdims=True)
    v = jnp.mean((x - m) * (x - m), axis=-1, keepdims=True)
    return (x - m) * lax.rsqrt(v + 1e-5) * g + b


def _transformer(x0, x1, wqkv, bqkv, wout, bout, wf1, bf1, wf2, bf2,
                 l1g, l1b, l2g, l2b):
    qkv0 = _dot(x0, wqkv) + bqkv
    qkv1 = _dot(x1, wqkv) + bqkv
    q0, k0, v0 = qkv0[:, 0:D], qkv0[:, D:2 * D], qkv0[:, 2 * D:3 * D]
    q1, k1, v1 = qkv1[:, 0:D], qkv1[:, D:2 * D], qkv1[:, 2 * D:3 * D]
    scale = 1.0 / (D ** 0.5)
    s00 = jnp.sum(q0 * k0, axis=-1, keepdims=True) * scale
    s01 = jnp.sum(q0 * k1, axis=-1, keepdims=True) * scale
    s10 = jnp.sum(q1 * k0, axis=-1, keepdims=True) * scale
    s11 = jnp.sum(q1 * k1, axis=-1, keepdims=True) * scale
    m0 = jnp.maximum(s00, s01)
    e00 = jnp.exp(s00 - m0)
    e01 = jnp.exp(s01 - m0)
    m1 = jnp.maximum(s10, s11)
    e10 = jnp.exp(s10 - m1)
    e11 = jnp.exp(s11 - m1)
    a0 = (e00 * v0 + e01 * v1) / (e00 + e01)
    a1 = (e10 * v0 + e11 * v1) / (e10 + e11)
    o0 = _dot(a0, wout) + bout
    o1 = _dot(a1, wout) + bout
    y0 = _ln(x0 + o0, l1g, l1b)
    y1 = _ln(x1 + o1, l1g, l1b)
    f0 = _dot(jnp.maximum(_dot(y0, wf1) + bf1, 0.0), wf2) + bf2
    f1 = _dot(jnp.maximum(_dot(y1, wf1) + bf1, 0.0), wf2) + bf2
    z0 = _ln(y0 + f0, l2g, l2b)
    z1 = _ln(y1 + f1, l2g, l2b)
    return 0.5 * (z0 + z1)


def _rb(shape=(BR, D)):
    return pl.BlockSpec(shape, lambda r: (r,) + (0,) * (len(shape) - 1))


def _full(shape):
    return pl.BlockSpec(shape, lambda r: (0,) * len(shape))


def _tc_prep(deg_parts, x, w):
    def body(dp, xr, wr, d0r, d1r, hpr):
        dp_ = dp[...]
        d0 = lax.rsqrt(dp_[0, 0][:, :1] + dp_[1, 0][:, :1] + 1.0)
        d1 = lax.rsqrt(dp_[0, 1][:, :1] + dp_[1, 1][:, :1] + 1.0)
        d0r[...] = d0
        d1r[...] = d1
        hpr[...] = _dot(xr[...], wr[...]) * d0

    return pl.pallas_call(
        body,
        grid=(R,),
        in_specs=[
            pl.BlockSpec((NC, 2, BR, D), lambda r: (0, 0, r, 0)),
            _rb(), _full((D, D)),
        ],
        out_specs=[_rb((BR, 1)), _rb((BR, 1)), _rb()],
        out_shape=[
            jax.ShapeDtypeStruct((N, 1), jnp.float32),
            jax.ShapeDtypeStruct((N, 1), jnp.float32),
            jax.ShapeDtypeStruct((N, D), jnp.float32),
        ],
    )(deg_parts, x, w)


def _tc_gcn_next(accp, hp, dinv, b, wn):
    def body(ar, hr, dr, br, wr, tr, nr):
        d = dr[...]
        tilde = jnp.maximum((_accsum(ar) + hr[...]) * d + br[...], 0.0)
        tr[...] = tilde
        nr[...] = _dot(tilde, wr[...]) * d

    return pl.pallas_call(
        body,
        grid=(R,),
        in_specs=[
            pl.BlockSpec((NC, BR, D), lambda r: (0, r, 0)),
            _rb(), _rb((BR, 1)), _full((1, D)), _full((D, D)),
        ],
        out_specs=[_rb(), _rb()],
        out_shape=[
            jax.ShapeDtypeStruct((N, D), jnp.float32),
            jax.ShapeDtypeStruct((N, D), jnp.float32),
        ],
    )(accp, hp, dinv, b, wn)


def _tc_gcn_bridge(accp, hp, dinv0, b, x1, w0, dinv1):
    def body(ar, hr, d0r, br, xr, wr, d1r, tr, gr):
        tilde = jnp.maximum((_accsum(ar) + hr[...]) * d0r[...] + br[...], 0.0)
        tr[...] = tilde
        gr[...] = _dot(xr[...], wr[...]) * d1r[...]

    return pl.pallas_call(
        body,
        grid=(R,),
        in_specs=[
            pl.BlockSpec((NC, BR, D), lambda r: (0, r, 0)),
            _rb(), _rb((BR, 1)), _full((1, D)), _rb(), _full((D, D)),
            _rb((BR, 1)),
        ],
        out_specs=[_rb(), _rb()],
        out_shape=[
            jax.ShapeDtypeStruct((N, D), jnp.float32),
            jax.ShapeDtypeStruct((N, D), jnp.float32),
        ],
    )(accp, hp, dinv0, b, x1, w0, dinv1)


def _tc_trans(accp, gp, dinv, bg, prev, tp, wn):
    def body(ar, gr, dr, bgr, pr, wqkv, bqkv, wout, bout, wf1, bf1, wf2, bf2,
             l1g, l1b, l2g, l2b, wnr, outr):
        d = dr[...]
        tilde = jnp.maximum((_accsum(ar) + gr[...]) * d + bgr[...], 0.0)
        h = _transformer(pr[...], tilde, wqkv[...], bqkv[...], wout[...],
                         bout[...], wf1[...], bf1[...], wf2[...], bf2[...],
                         l1g[...], l1b[...], l2g[...], l2b[...])
        outr[...] = _dot(h, wnr[...]) * d

    return pl.pallas_call(
        body,
        grid=(R,),
        in_specs=[
            pl.BlockSpec((NC, BR, D), lambda r: (0, r, 0)),
            _rb(), _rb((BR, 1)), _full((1, D)), _rb(),
            _full((D, 3 * D)), _full((1, 3 * D)), _full((D, D)),
            _full((1, D)), _full((D, 2048)), _full((1, 2048)),
            _full((2048, D)), _full((1, D)),
            _full((1, D)), _full((1, D)), _full((1, D)), _full((1, D)),
            _full((D, D)),
        ],
        out_specs=[_rb()],
        out_shape=[jax.ShapeDtypeStruct((N, D), jnp.float32)],
    )(accp, gp, dinv, bg, prev, *tp, wn)[0]


def _tc_final(accp, gp, dinv, bg, prev, tp, mlp_wt, mlp_b, out_wt, out_b):
    def body(ar, gr, dr, bgr, pr, wqkv, bqkv, wout, bout, wf1, bf1, wf2, bf2,
             l1g, l1b, l2g, l2b, mwr, mbr, owr, obr, outr, pacc):
        r = pl.program_id(0)
        d = dr[...]
        tilde = jnp.maximum((_accsum(ar) + gr[...]) * d + bgr[...], 0.0)
        h = _transformer(pr[...], tilde, wqkv[...], bqkv[...], wout[...],
                         bout[...], wf1[...], bf1[...], wf2[...], bf2[...],
                         l1g[...], l1b[...], l2g[...], l2b[...])

        @pl.when(r == 0)
        def _():
            pacc[...] = jnp.zeros_like(pacc)

        pacc[...] += jnp.sum(h, axis=0, keepdims=True)
        pooled = pacc[...] * (1.0 / N)
        hm = jnp.maximum(_dot(pooled, mwr[...]) + mbr[...], 0.0)
        outr[...] = _dot(hm, owr[...]) + obr[...]

    return pl.pallas_call(
        body,
        grid=(R,),
        in_specs=[
            pl.BlockSpec((NC, BR, D), lambda r: (0, r, 0)),
            _rb(), _rb((BR, 1)), _full((1, D)), _rb(),
            _full((D, 3 * D)), _full((1, 3 * D)), _full((D, D)),
            _full((1, D)), _full((D, 2048)), _full((1, 2048)),
            _full((2048, D)), _full((1, D)),
            _full((1, D)), _full((1, D)), _full((1, D)), _full((1, D)),
            _full((D, D)), _full((1, D)), _full((D, 16)), _full((1, 16)),
        ],
        out_specs=[_full((1, 16))],
        out_shape=[jax.ShapeDtypeStruct((1, 16), jnp.float32)],
        scratch_shapes=[pltpu.VMEM((1, D), jnp.float32)],
    )(accp, gp, dinv, bg, prev, *tp, mlp_wt, mlp_b, out_wt, out_b)[0]


def _tparams(lp):
    return (
        lp['in_proj_w'].T, lp['in_proj_b'][None, :],
        lp['out_proj_w'].T, lp['out_proj_b'][None, :],
        lp['lin1_w'].T, lp['lin1_b'][None, :],
        lp['lin2_w'].T, lp['lin2_b'][None, :],
        lp['ln1_g'][None, :], lp['ln1_b'][None, :],
        lp['ln2_g'][None, :], lp['ln2_b'][None, :],
    )


def kernel(x0, x1, edge_index0, edge_index1, params):
    def prep_idx(row, fill):
        r = row.astype(jnp.int32).reshape(NW, EP)
        return jnp.pad(r, ((0, 0), (0, PAD)),
                       constant_values=fill).reshape(NW, CH, K)

    src0 = prep_idx(edge_index0[0], 0)
    dst0 = prep_idx(edge_index0[1], N)
    src1 = prep_idx(edge_index1[0], 0)
    dst1 = prep_idx(edge_index1[1], N)

    l0, l1 = params['layers']
    w0, b0 = l0['gcn_W'], l0['gcn_b'][None, :]
    w1, b1 = l1['gcn_W'], l1['gcn_b'][None, :]

    deg_parts = _sc_degree(dst0, dst1)
    dinv0, dinv1, h0p = _tc_prep(deg_parts, x0, w0)

    pk0 = (src0 << 14) | dst0
    pk1 = (src1 << 14) | dst1

    acc = _sc_scatter(h0p, pk0)
    prev0, h1p = _tc_gcn_next(acc, h0p, dinv0, b0, w1)
    acc = _sc_scatter(h1p, pk0)
    prev1, g0p = _tc_gcn_bridge(acc, h1p, dinv0, b1, x1, w0, dinv1)

    acc = _sc_scatter(g0p, pk1)
    g1p = _tc_trans(acc, g0p, dinv1, b0, prev0, _tparams(l0), w1)
    acc = _sc_scatter(g1p, pk1)
    return _tc_final(acc, g1p, dinv1, b1, prev1, _tparams(l1),
                     params['mlp_w'].T, params['mlp_b'][None, :],
                     params['out_w'].T, params['out_b'][None, :])
